# baseline (speedup 1.0000x reference)
import functools

import jax
import jax.numpy as jnp
from jax import lax
from jax.experimental import pallas as pl
from jax.experimental.pallas import tpu as pltpu
from jax.experimental.pallas import tpu_sc as plsc

N = 10000
E = 320000
H = 128
G = 64
L = 3

_NC = 2
_NS = 16
_NW = _NC * _NS
_ZSTEP = 624
_ZROWS = 640

_K = 40
_CPT = E // (_NW * _K)

_KD = 128
_DCHUNKS = E // _KD
_DPER = -(-_DCHUNKS // _NW)

def _sc_mesh():
    return plsc.VectorSubcoreMesh(core_axis_name="c", subcore_axis_name="s")


def _sc_message(A, B, C3, src, dst, zrows):
    @functools.partial(
        pl.kernel,
        out_type=jax.ShapeDtypeStruct((_NC, N, H), jnp.float32),
        mesh=_sc_mesh(),
        scratch_types=[
            [pltpu.VMEM((2 * _K,), jnp.int32)] * 2,
            [pltpu.VMEM((2 * _K,), jnp.int32)] * 2,
            [pltpu.VMEM((_K,), jnp.int32)] * 2,
            [pltpu.VMEM((_K, H), jnp.float32)] * 2,
            [pltpu.VMEM((_K, H), jnp.float32)] * 2,
            [pltpu.VMEM((_K, H), jnp.float32)] * 2,
            pltpu.VMEM_SHARED((N, H), jnp.float32),
            [pltpu.SemaphoreType.DMA] * 2,
            [pltpu.SemaphoreType.DMA] * 2,
            [pltpu.SemaphoreType.DMA] * 2,
            pltpu.SemaphoreType.DMA,
            [pltpu.SemaphoreType.DMA] * 2,
        ],
    )
    def k(a_hbm, b_hbm, c_hbm, src_hbm, dst_hbm, z_hbm, out_hbm,
          idx_d, idx_s, ring_d, buf_a, buf_b, buf_c, accum,
          sem_a, sem_b, sem_c, sem_i, sem_s):
        cid = lax.axis_index("c")
        sid = lax.axis_index("s")
        wid = sid * _NC + cid
        row0 = sid * _ZSTEP
        base_e = wid * _CPT * _K
        pltpu.sync_copy(z_hbm, accum.at[pl.ds(row0, _ZROWS)])
        pltpu.sync_copy(dst_hbm.at[pl.ds(base_e, 2 * _K)], idx_d[0])
        pltpu.sync_copy(src_hbm.at[pl.ds(base_e, 2 * _K)], idx_s[0])
        plsc.subcore_barrier()

        def scat_drain(s):
            pltpu.make_async_copy(buf_c[s], accum.at[ring_d[s]],
                                  sem_s[s]).wait()

        def issue(s, half, ip, j):
            if isinstance(j, int):
                if j >= 2:
                    scat_drain(s)
            else:
                @pl.when(j >= 2)
                def _():
                    scat_drain(s)
            sl = pl.ds(half * _K, _K)
            pltpu.async_copy(c_hbm.at[wid * _CPT + j], buf_c[s], sem_c[s])
            pltpu.async_copy(a_hbm.at[idx_d[ip].at[sl]], buf_a[s], sem_a[s])
            pltpu.async_copy(b_hbm.at[idx_s[ip].at[sl]], buf_b[s], sem_b[s])

        def finish(s, half, ip):
            for off in (0, 16, _K - 16):
                ring_d[s][pl.ds(off, 16)] = idx_d[ip][pl.ds(half * _K + off, 16)]
            pltpu.make_async_copy(c_hbm.at[0], buf_c[s], sem_c[s]).wait()
            pltpu.make_async_copy(c_hbm.at[0], buf_a[s], sem_a[s]).wait()
            pltpu.make_async_copy(c_hbm.at[0], buf_b[s], sem_b[s]).wait()

            def row(r, c2):
                for q in range(H // 16):
                    sl = pl.ds(q * 16, 16)
                    v = buf_c[s][r, sl] + buf_a[s][r, sl] + buf_b[s][r, sl]
                    buf_c[s][r, sl] = jnp.maximum(v, 0.0)
                return c2

            lax.fori_loop(0, _K, row, 0)
            pltpu.async_copy(buf_c[s], accum.at[ring_d[s]], sem_s[s], add=True)

        def idx_issue(ip, p):
            nb = base_e + p * 2 * _K
            pltpu.async_copy(dst_hbm.at[pl.ds(nb, 2 * _K)], idx_d[ip], sem_i)
            pltpu.async_copy(src_hbm.at[pl.ds(nb, 2 * _K)], idx_s[ip], sem_i)

        def idx_wait(ip):
            pltpu.make_async_copy(dst_hbm.at[pl.ds(0, 2 * _K)],
                                  idx_d[ip], sem_i).wait()
            pltpu.make_async_copy(dst_hbm.at[pl.ds(0, 2 * _K)],
                                  idx_s[ip], sem_i).wait()

        def block(p, ip):
            idx_issue(1 - ip, p + 1)
            issue(1, 1, ip, 2 * p + 1)
            finish(0, 0, ip)
            idx_wait(1 - ip)
            issue(0, 0, 1 - ip, 2 * p + 2)
            finish(1, 1, ip)

        issue(0, 0, 0, 0)

        def qloop(q, carry):
            block(2 * q, 0)
            block(2 * q + 1, 1)
            return carry

        _PAIRS = _CPT // 2
        lax.fori_loop(0, (_PAIRS - 1) // 2, qloop, 0)
        issue(1, 1, 0, _CPT - 1)
        finish(0, 0, 0)
        finish(1, 1, 0)
        scat_drain(0)
        scat_drain(1)
        plsc.subcore_barrier()
        pltpu.sync_copy(accum.at[pl.ds(row0, _ZROWS)],
                        out_hbm.at[cid, pl.ds(row0, _ZROWS)])

    return k(A, B, C3, src, dst, zrows)


def _sc_degree(dst, zrows):
    @functools.partial(
        pl.kernel,
        out_type=jax.ShapeDtypeStruct((_NC, N, H), jnp.float32),
        mesh=_sc_mesh(),
        scratch_types=[
            pltpu.VMEM((_KD,), jnp.int32),
            pltpu.VMEM((_KD, H), jnp.float32),
            pltpu.VMEM_SHARED((N, H), jnp.float32),
        ],
    )
    def k(dst_hbm, z_hbm, out_hbm, idx_d, ones_b, accum):
        cid = lax.axis_index("c")
        sid = lax.axis_index("s")
        wid = sid * _NC + cid
        row0 = sid * _ZSTEP

        def fill(r, c2):
            for q in range(H // 16):
                ones_b[r, pl.ds(q * 16, 16)] = jnp.full((16,), 1.0, jnp.float32)
            return c2

        lax.fori_loop(0, _KD, fill, 0)
        pltpu.sync_copy(z_hbm, accum.at[pl.ds(row0, _ZROWS)])
        plsc.subcore_barrier()

        def chunk(j, carry):
            cidx = j * _NW + wid

            @pl.when(cidx < _DCHUNKS)
            def _():
                base = cidx * _KD
                pltpu.sync_copy(dst_hbm.at[pl.ds(base, _KD)], idx_d)
                pltpu.sync_copy(ones_b, accum.at[idx_d], add=True)

            return carry

        lax.fori_loop(0, _DPER, chunk, 0)
        plsc.subcore_barrier()
        pltpu.sync_copy(accum.at[pl.ds(row0, _ZROWS)],
                        out_hbm.at[cid, pl.ds(row0, _ZROWS)])

    return k(dst, zrows)


def _dot(a, b):
    return jnp.dot(a, b, preferred_element_type=jnp.float32)


_BN = 2000
_BE = 4000


def _node_encode_body(x_ref, nw_ref, nb_ref, mw1_ref, mw2_ref, mb_ref,
                      h_ref, a_ref, b_ref):
    h = jnp.maximum(_dot(x_ref[...], nw_ref[...]) + nb_ref[...], 0.0)
    h_ref[...] = h
    a_ref[...] = _dot(h, mw1_ref[...]) + mb_ref[...]
    b_ref[...] = _dot(h, mw2_ref[...])


def _tc_node_encode(x, nw, nb, mw1, mw2, mb):
    f32 = jnp.float32
    return pl.pallas_call(
        _node_encode_body,
        grid=(N // _BN,),
        in_specs=[
            pl.BlockSpec((_BN, H), lambda i: (i, 0)),
            pl.BlockSpec((H, H), lambda i: (0, 0)),
            pl.BlockSpec((1, H), lambda i: (0, 0)),
            pl.BlockSpec((H, H), lambda i: (0, 0)),
            pl.BlockSpec((H, H), lambda i: (0, 0)),
            pl.BlockSpec((1, H), lambda i: (0, 0)),
        ],
        out_specs=[
            pl.BlockSpec((_BN, H), lambda i: (i, 0)),
            pl.BlockSpec((_BN, H), lambda i: (i, 0)),
            pl.BlockSpec((_BN, H), lambda i: (i, 0)),
        ],
        out_shape=[
            jax.ShapeDtypeStruct((N, H), f32),
            jax.ShapeDtypeStruct((N, H), f32),
            jax.ShapeDtypeStruct((N, H), f32),
        ],
    )(x, nw, nb, mw1, mw2, mb)


def _edge_encode_body(ea_ref, ew_ref, eb_ref, mw3_ref, out_ref, c_ref):
    t = jnp.maximum(_dot(ea_ref[...], ew_ref[...]) + eb_ref[...], 0.0)
    out_ref[...] = t
    c_ref[...] = _dot(t, mw3_ref[...])


def _tc_edge_encode(ea, ew, eb, mw3):
    f32 = jnp.float32
    d_in = ea.shape[1]
    return pl.pallas_call(
        _edge_encode_body,
        grid=(E // _BE,),
        in_specs=[
            pl.BlockSpec((_BE, d_in), lambda i: (i, 0)),
            pl.BlockSpec((d_in, H), lambda i: (0, 0)),
            pl.BlockSpec((1, H), lambda i: (0, 0)),
            pl.BlockSpec((H, H), lambda i: (0, 0)),
        ],
        out_specs=[
            pl.BlockSpec((_BE, H), lambda i: (i, 0)),
            pl.BlockSpec((_BE, H), lambda i: (i, 0)),
        ],
        out_shape=[
            jax.ShapeDtypeStruct((E, H), f32),
            jax.ShapeDtypeStruct((E, H), f32),
        ],
    )(ea, ew, eb, mw3)


def _edge_update_body(ea_ref, elw_ref, elb_ref, mw3_ref, out_ref, c_ref):
    t = jnp.maximum(_dot(ea_ref[...], elw_ref[...]) + elb_ref[...], 0.0)
    out_ref[...] = t
    c_ref[...] = _dot(t, mw3_ref[...])


def _tc_edge_update(ea, elw, elb, mw3):
    f32 = jnp.float32
    return pl.pallas_call(
        _edge_update_body,
        grid=(E // _BE,),
        in_specs=[
            pl.BlockSpec((_BE, H), lambda i: (i, 0)),
            pl.BlockSpec((H, H), lambda i: (0, 0)),
            pl.BlockSpec((1, H), lambda i: (0, 0)),
            pl.BlockSpec((H, H), lambda i: (0, 0)),
        ],
        out_specs=[
            pl.BlockSpec((_BE, H), lambda i: (i, 0)),
            pl.BlockSpec((_BE, H), lambda i: (i, 0)),
        ],
        out_shape=[
            jax.ShapeDtypeStruct((E, H), f32),
            jax.ShapeDtypeStruct((E, H), f32),
        ],
    )(ea, elw, elb, mw3)


def _node_update_core(h_ref, s_ref, d_ref, uw1_ref, uw2_ref, ub_ref,
                      nlw_ref, nlb_ref):
    s = s_ref[0] + s_ref[1]
    deg = jnp.maximum(d_ref[0, :, :1] + d_ref[1, :, :1], 1.0)
    aggr = s / deg
    t = jnp.maximum(
        _dot(h_ref[...], uw1_ref[...]) + _dot(aggr, uw2_ref[...]) + ub_ref[...],
        0.0)
    return jnp.maximum(_dot(t, nlw_ref[...]) + nlb_ref[...], 0.0)


def _node_update_body(h_ref, s_ref, d_ref, uw1_ref, uw2_ref, ub_ref,
                      nlw_ref, nlb_ref, mw1_ref, mw2_ref, mb_ref,
                      h_out, a_out, b_out):
    h2 = _node_update_core(h_ref, s_ref, d_ref, uw1_ref, uw2_ref, ub_ref,
                           nlw_ref, nlb_ref)
    h_out[...] = h2
    a_out[...] = _dot(h2, mw1_ref[...]) + mb_ref[...]
    b_out[...] = _dot(h2, mw2_ref[...])


def _node_update_last_body(h_ref, s_ref, d_ref, uw1_ref, uw2_ref, ub_ref,
                           nlw_ref, nlb_ref, h_out):
    h_out[...] = _node_update_core(h_ref, s_ref, d_ref, uw1_ref, uw2_ref,
                                   ub_ref, nlw_ref, nlb_ref)


def _nu_in_specs():
    return [
        pl.BlockSpec((_BN, H), lambda i: (i, 0)),
        pl.BlockSpec((_NC, _BN, H), lambda i: (0, i, 0)),
        pl.BlockSpec((_NC, _BN, H), lambda i: (0, i, 0)),
        pl.BlockSpec((H, H), lambda i: (0, 0)),
        pl.BlockSpec((H, H), lambda i: (0, 0)),
        pl.BlockSpec((1, H), lambda i: (0, 0)),
        pl.BlockSpec((H, H), lambda i: (0, 0)),
        pl.BlockSpec((1, H), lambda i: (0, 0)),
    ]


def _tc_node_update(h, s, d, uw1, uw2, ub, nlw, nlb, mw1, mw2, mb):
    f32 = jnp.float32
    return pl.pallas_call(
        _node_update_body,
        grid=(N // _BN,),
        in_specs=_nu_in_specs() + [
            pl.BlockSpec((H, H), lambda i: (0, 0)),
            pl.BlockSpec((H, H), lambda i: (0, 0)),
            pl.BlockSpec((1, H), lambda i: (0, 0)),
        ],
        out_specs=[
            pl.BlockSpec((_BN, H), lambda i: (i, 0)),
            pl.BlockSpec((_BN, H), lambda i: (i, 0)),
            pl.BlockSpec((_BN, H), lambda i: (i, 0)),
        ],
        out_shape=[
            jax.ShapeDtypeStruct((N, H), f32),
            jax.ShapeDtypeStruct((N, H), f32),
            jax.ShapeDtypeStruct((N, H), f32),
        ],
    )(h, s, d, uw1, uw2, ub, nlw, nlb, mw1, mw2, mb)


def _tc_node_update_last(h, s, d, uw1, uw2, ub, nlw, nlb):
    return pl.pallas_call(
        _node_update_last_body,
        grid=(N // _BN,),
        in_specs=_nu_in_specs(),
        out_specs=pl.BlockSpec((_BN, H), lambda i: (i, 0)),
        out_shape=jax.ShapeDtypeStruct((N, H), jnp.float32),
    )(h, s, d, uw1, uw2, ub, nlw, nlb)


_BP = 400


def _pool_head_body(h_ref, b_ref, hw_ref, hb_ref, out_ref, pooled, cnt):
    i = pl.program_id(0)

    @pl.when(i == 0)
    def _():
        pooled[...] = jnp.zeros_like(pooled)
        cnt[...] = jnp.zeros_like(cnt)

    gids = lax.broadcasted_iota(jnp.int32, (_BP, G), 1)
    onehot = jnp.where(b_ref[...] == gids, 1.0, 0.0).astype(jnp.float32)
    dn = (((0,), (0,)), ((), ()))
    pooled[...] += lax.dot_general(onehot, h_ref[...], dn,
                                   preferred_element_type=jnp.float32)
    cnt[...] += lax.dot_general(onehot, jnp.ones((_BP, H), jnp.float32), dn,
                                preferred_element_type=jnp.float32)

    @pl.when(i == pl.num_programs(0) - 1)
    def _():
        pm = pooled[...] / jnp.maximum(cnt[...], 1.0)
        out_ref[...] = _dot(pm, hw_ref[...]) + hb_ref[...]


def _tc_pool_head(h, batch2d, hw, hb):
    return pl.pallas_call(
        _pool_head_body,
        grid=(N // _BP,),
        in_specs=[
            pl.BlockSpec((_BP, H), lambda i: (i, 0)),
            pl.BlockSpec((_BP, 1), lambda i: (i, 0)),
            pl.BlockSpec((H, 1), lambda i: (0, 0)),
            pl.BlockSpec((1, 1), lambda i: (0, 0)),
        ],
        out_specs=pl.BlockSpec((G, 1), lambda i: (0, 0)),
        out_shape=jax.ShapeDtypeStruct((G, 1), jnp.float32),
        scratch_shapes=[
            pltpu.VMEM((G, H), jnp.float32),
            pltpu.VMEM((G, H), jnp.float32),
        ],
    )(h, batch2d, hw, hb)


def kernel(x, edge_index, edge_attr, batch, node_W, node_b, edge_W, edge_b,
           M_W, M_b, U_W, U_b, nl_W, nl_b, el_W, el_b, head_W, head_b):
    src = edge_index[0]
    dst = edge_index[1]
    mw1 = M_W[:, :H, :]
    mw2 = M_W[:, H:2 * H, :]
    mw3 = M_W[:, 2 * H:, :]
    uw1 = U_W[:, :H, :]
    uw2 = U_W[:, H:, :]
    zrows = jnp.zeros((_ZROWS, H), jnp.float32)

    h, a_tab, b_tab = _tc_node_encode(
        x, node_W, node_b.reshape(1, H), mw1[0], mw2[0], M_b[0].reshape(1, H))
    dpart = _sc_degree(dst, zrows)

    ea, c0 = _tc_edge_encode(edge_attr, edge_W, edge_b.reshape(1, H), mw3[0])
    cs = [c0]
    for i in range(L - 1):
        ea, ci = _tc_edge_update(ea, el_W[i], el_b[i].reshape(1, H),
                                 mw3[i + 1])
        cs.append(ci)

    for i in range(L):
        s = _sc_message(a_tab, b_tab, cs[i].reshape(_NW * _CPT, _K, H),
                        src, dst, zrows)
        if i + 1 < L:
            h, a_tab, b_tab = _tc_node_update(
                h, s, dpart, uw1[i], uw2[i], U_b[i].reshape(1, H),
                nl_W[i], nl_b[i].reshape(1, H),
                mw1[i + 1], mw2[i + 1], M_b[i + 1].reshape(1, H))
        else:
            h = _tc_node_update_last(
                h, s, dpart, uw1[i], uw2[i], U_b[i].reshape(1, H),
                nl_W[i], nl_b[i].reshape(1, H))

    return _tc_pool_head(h, batch.reshape(N, 1), head_W, head_b.reshape(1, 1))


# speedup vs baseline: 1.0787x; 1.0787x over previous
import functools

import jax
import jax.numpy as jnp
from jax import lax
from jax.experimental import pallas as pl
from jax.experimental.pallas import tpu as pltpu
from jax.experimental.pallas import tpu_sc as plsc

N = 10000
E = 320000
H = 128
G = 64
L = 3

_NC = 2
_NS = 16
_NW = _NC * _NS
_ZSTEP = 624
_ZROWS = 640

_K = 40
_CPT = E // (_NW * _K)

_KD = 128
_DCHUNKS = E // _KD
_DPER = -(-_DCHUNKS // _NW)

def _sc_mesh():
    return plsc.VectorSubcoreMesh(core_axis_name="c", subcore_axis_name="s")


def _sc_message(A, B, C3, src, dst, zrows):
    @functools.partial(
        pl.kernel,
        out_type=jax.ShapeDtypeStruct((_NC, N, H), jnp.float32),
        mesh=_sc_mesh(),
        scratch_types=[
            [pltpu.VMEM((2 * _K,), jnp.int32)] * 2,
            [pltpu.VMEM((2 * _K,), jnp.int32)] * 2,
            [pltpu.VMEM((_K,), jnp.int32)] * 2,
            [pltpu.VMEM((_K, H), jnp.float32)] * 2,
            [pltpu.VMEM((_K, H), jnp.float32)] * 2,
            [pltpu.VMEM((_K, H), jnp.float32)] * 2,
            pltpu.VMEM_SHARED((N, H), jnp.float32),
            [pltpu.SemaphoreType.DMA] * 2,
            [pltpu.SemaphoreType.DMA] * 2,
            [pltpu.SemaphoreType.DMA] * 2,
            pltpu.SemaphoreType.DMA,
            [pltpu.SemaphoreType.DMA] * 2,
        ],
    )
    def k(a_hbm, b_hbm, c_hbm, src_hbm, dst_hbm, z_hbm, out_hbm,
          idx_d, idx_s, ring_d, buf_a, buf_b, buf_c, accum,
          sem_a, sem_b, sem_c, sem_i, sem_s):
        cid = lax.axis_index("c")
        sid = lax.axis_index("s")
        wid = sid * _NC + cid
        row0 = sid * _ZSTEP
        base_e = wid * _CPT * _K
        pltpu.sync_copy(z_hbm, accum.at[pl.ds(row0, _ZROWS)])
        pltpu.sync_copy(dst_hbm.at[pl.ds(base_e, 2 * _K)], idx_d[0])
        pltpu.sync_copy(src_hbm.at[pl.ds(base_e, 2 * _K)], idx_s[0])
        plsc.subcore_barrier()

        def scat_drain(s):
            pltpu.make_async_copy(buf_c[s], accum.at[ring_d[s]],
                                  sem_s[s]).wait()

        def issue(s, half, ip, j):
            if isinstance(j, int):
                if j >= 2:
                    scat_drain(s)
            else:
                @pl.when(j >= 2)
                def _():
                    scat_drain(s)
            sl = pl.ds(half * _K, _K)
            pltpu.async_copy(c_hbm.at[wid * _CPT + j], buf_c[s], sem_c[s])
            pltpu.async_copy(a_hbm.at[idx_d[ip].at[sl]], buf_a[s], sem_a[s])
            pltpu.async_copy(b_hbm.at[idx_s[ip].at[sl]], buf_b[s], sem_b[s])

        def finish(s, half, ip):
            for off in (0, 16, _K - 16):
                ring_d[s][pl.ds(off, 16)] = idx_d[ip][pl.ds(half * _K + off, 16)]
            pltpu.make_async_copy(c_hbm.at[0], buf_c[s], sem_c[s]).wait()
            pltpu.make_async_copy(c_hbm.at[0], buf_a[s], sem_a[s]).wait()
            pltpu.make_async_copy(c_hbm.at[0], buf_b[s], sem_b[s]).wait()

            def row(r, c2):
                for q in range(H // 16):
                    sl = pl.ds(q * 16, 16)
                    v = buf_c[s][r, sl] + buf_a[s][r, sl] + buf_b[s][r, sl]
                    buf_c[s][r, sl] = jnp.maximum(v, 0.0)
                return c2

            lax.fori_loop(0, _K, row, 0)
            pltpu.async_copy(buf_c[s], accum.at[ring_d[s]], sem_s[s], add=True)

        def idx_issue(ip, p):
            nb = base_e + p * 2 * _K
            pltpu.async_copy(dst_hbm.at[pl.ds(nb, 2 * _K)], idx_d[ip], sem_i)
            pltpu.async_copy(src_hbm.at[pl.ds(nb, 2 * _K)], idx_s[ip], sem_i)

        def idx_wait(ip):
            pltpu.make_async_copy(dst_hbm.at[pl.ds(0, 2 * _K)],
                                  idx_d[ip], sem_i).wait()
            pltpu.make_async_copy(dst_hbm.at[pl.ds(0, 2 * _K)],
                                  idx_s[ip], sem_i).wait()

        def block(p, ip):
            idx_issue(1 - ip, p + 1)
            issue(1, 1, ip, 2 * p + 1)
            finish(0, 0, ip)
            idx_wait(1 - ip)
            issue(0, 0, 1 - ip, 2 * p + 2)
            finish(1, 1, ip)

        issue(0, 0, 0, 0)

        def qloop(q, carry):
            block(2 * q, 0)
            block(2 * q + 1, 1)
            return carry

        _PAIRS = _CPT // 2
        lax.fori_loop(0, (_PAIRS - 1) // 2, qloop, 0)
        issue(1, 1, 0, _CPT - 1)
        finish(0, 0, 0)
        finish(1, 1, 0)
        scat_drain(0)
        scat_drain(1)
        plsc.subcore_barrier()
        pltpu.sync_copy(accum.at[pl.ds(row0, _ZROWS)],
                        out_hbm.at[cid, pl.ds(row0, _ZROWS)])

    return k(A, B, C3, src, dst, zrows)


def _sc_degree(dst, zrows):
    @functools.partial(
        pl.kernel,
        out_type=jax.ShapeDtypeStruct((_NC, N, H), jnp.float32),
        mesh=_sc_mesh(),
        scratch_types=[
            pltpu.VMEM((_KD,), jnp.int32),
            pltpu.VMEM((_KD, H), jnp.float32),
            pltpu.VMEM_SHARED((N, H), jnp.float32),
        ],
    )
    def k(dst_hbm, z_hbm, out_hbm, idx_d, ones_b, accum):
        cid = lax.axis_index("c")
        sid = lax.axis_index("s")
        wid = sid * _NC + cid
        row0 = sid * _ZSTEP

        def fill(r, c2):
            for q in range(H // 16):
                ones_b[r, pl.ds(q * 16, 16)] = jnp.full((16,), 1.0, jnp.float32)
            return c2

        lax.fori_loop(0, _KD, fill, 0)
        pltpu.sync_copy(z_hbm, accum.at[pl.ds(row0, _ZROWS)])
        plsc.subcore_barrier()

        def chunk(j, carry):
            cidx = j * _NW + wid

            @pl.when(cidx < _DCHUNKS)
            def _():
                base = cidx * _KD
                pltpu.sync_copy(dst_hbm.at[pl.ds(base, _KD)], idx_d)
                pltpu.sync_copy(ones_b, accum.at[idx_d], add=True)

            return carry

        lax.fori_loop(0, _DPER, chunk, 0)
        plsc.subcore_barrier()
        pltpu.sync_copy(accum.at[pl.ds(row0, _ZROWS)],
                        out_hbm.at[cid, pl.ds(row0, _ZROWS)])

    return k(dst, zrows)


def _dot(a, b):
    return jnp.dot(a, b, preferred_element_type=jnp.float32)


_BN = 2000
_BE = 4000


def _node_encode_body(x_ref, nw_ref, nb_ref, mw1_ref, mw2_ref, mb_ref,
                      h_ref, a_ref, b_ref):
    h = jnp.maximum(_dot(x_ref[...], nw_ref[...]) + nb_ref[...], 0.0)
    h_ref[...] = h
    a_ref[...] = _dot(h, mw1_ref[...]) + mb_ref[...]
    b_ref[...] = _dot(h, mw2_ref[...])


def _tc_node_encode(x, nw, nb, mw1, mw2, mb):
    f32 = jnp.float32
    return pl.pallas_call(
        _node_encode_body,
        grid=(N // _BN,),
        in_specs=[
            pl.BlockSpec((_BN, H), lambda i: (i, 0)),
            pl.BlockSpec((H, H), lambda i: (0, 0)),
            pl.BlockSpec((1, H), lambda i: (0, 0)),
            pl.BlockSpec((H, H), lambda i: (0, 0)),
            pl.BlockSpec((H, H), lambda i: (0, 0)),
            pl.BlockSpec((1, H), lambda i: (0, 0)),
        ],
        out_specs=[
            pl.BlockSpec((_BN, H), lambda i: (i, 0)),
            pl.BlockSpec((_BN, H), lambda i: (i, 0)),
            pl.BlockSpec((_BN, H), lambda i: (i, 0)),
        ],
        out_shape=[
            jax.ShapeDtypeStruct((N, H), f32),
            jax.ShapeDtypeStruct((N, H), f32),
            jax.ShapeDtypeStruct((N, H), f32),
        ],
    )(x, nw, nb, mw1, mw2, mb)


def _edge_encode_body(ea_ref, ew_ref, eb_ref, mw3_ref, out_ref, c_ref):
    t = jnp.maximum(_dot(ea_ref[...], ew_ref[...]) + eb_ref[...], 0.0)
    out_ref[...] = t.astype(jnp.bfloat16)
    c_ref[...] = _dot(t, mw3_ref[...])


def _tc_edge_encode(ea, ew, eb, mw3):
    f32 = jnp.float32
    d_in = ea.shape[1]
    return pl.pallas_call(
        _edge_encode_body,
        grid=(E // _BE,),
        in_specs=[
            pl.BlockSpec((_BE, d_in), lambda i: (i, 0)),
            pl.BlockSpec((d_in, H), lambda i: (0, 0)),
            pl.BlockSpec((1, H), lambda i: (0, 0)),
            pl.BlockSpec((H, H), lambda i: (0, 0)),
        ],
        out_specs=[
            pl.BlockSpec((_BE, H), lambda i: (i, 0)),
            pl.BlockSpec((_BE, H), lambda i: (i, 0)),
        ],
        out_shape=[
            jax.ShapeDtypeStruct((E, H), jnp.bfloat16),
            jax.ShapeDtypeStruct((E, H), f32),
        ],
    )(ea, ew, eb, mw3)


def _edge_update_body(ea_ref, elw_ref, elb_ref, mw3_ref, out_ref, c_ref):
    t = jnp.maximum(_dot(ea_ref[...].astype(jnp.float32), elw_ref[...])
                    + elb_ref[...], 0.0)
    out_ref[...] = t.astype(jnp.bfloat16)
    c_ref[...] = _dot(t, mw3_ref[...])


def _tc_edge_update(ea, elw, elb, mw3):
    f32 = jnp.float32
    return pl.pallas_call(
        _edge_update_body,
        grid=(E // _BE,),
        in_specs=[
            pl.BlockSpec((_BE, H), lambda i: (i, 0)),
            pl.BlockSpec((H, H), lambda i: (0, 0)),
            pl.BlockSpec((1, H), lambda i: (0, 0)),
            pl.BlockSpec((H, H), lambda i: (0, 0)),
        ],
        out_specs=[
            pl.BlockSpec((_BE, H), lambda i: (i, 0)),
            pl.BlockSpec((_BE, H), lambda i: (i, 0)),
        ],
        out_shape=[
            jax.ShapeDtypeStruct((E, H), jnp.bfloat16),
            jax.ShapeDtypeStruct((E, H), f32),
        ],
    )(ea, elw, elb, mw3)


def _node_update_core(h_ref, s_ref, d_ref, uw1_ref, uw2_ref, ub_ref,
                      nlw_ref, nlb_ref):
    s = s_ref[0] + s_ref[1]
    deg = jnp.maximum(d_ref[0, :, :1] + d_ref[1, :, :1], 1.0)
    aggr = s / deg
    t = jnp.maximum(
        _dot(h_ref[...], uw1_ref[...]) + _dot(aggr, uw2_ref[...]) + ub_ref[...],
        0.0)
    return jnp.maximum(_dot(t, nlw_ref[...]) + nlb_ref[...], 0.0)


def _node_update_body(h_ref, s_ref, d_ref, uw1_ref, uw2_ref, ub_ref,
                      nlw_ref, nlb_ref, mw1_ref, mw2_ref, mb_ref,
                      h_out, a_out, b_out):
    h2 = _node_update_core(h_ref, s_ref, d_ref, uw1_ref, uw2_ref, ub_ref,
                           nlw_ref, nlb_ref)
    h_out[...] = h2
    a_out[...] = _dot(h2, mw1_ref[...]) + mb_ref[...]
    b_out[...] = _dot(h2, mw2_ref[...])


def _node_update_last_body(h_ref, s_ref, d_ref, uw1_ref, uw2_ref, ub_ref,
                           nlw_ref, nlb_ref, h_out):
    h_out[...] = _node_update_core(h_ref, s_ref, d_ref, uw1_ref, uw2_ref,
                                   ub_ref, nlw_ref, nlb_ref)


def _nu_in_specs():
    return [
        pl.BlockSpec((_BN, H), lambda i: (i, 0)),
        pl.BlockSpec((_NC, _BN, H), lambda i: (0, i, 0)),
        pl.BlockSpec((_NC, _BN, H), lambda i: (0, i, 0)),
        pl.BlockSpec((H, H), lambda i: (0, 0)),
        pl.BlockSpec((H, H), lambda i: (0, 0)),
        pl.BlockSpec((1, H), lambda i: (0, 0)),
        pl.BlockSpec((H, H), lambda i: (0, 0)),
        pl.BlockSpec((1, H), lambda i: (0, 0)),
    ]


def _tc_node_update(h, s, d, uw1, uw2, ub, nlw, nlb, mw1, mw2, mb):
    f32 = jnp.float32
    return pl.pallas_call(
        _node_update_body,
        grid=(N // _BN,),
        in_specs=_nu_in_specs() + [
            pl.BlockSpec((H, H), lambda i: (0, 0)),
            pl.BlockSpec((H, H), lambda i: (0, 0)),
            pl.BlockSpec((1, H), lambda i: (0, 0)),
        ],
        out_specs=[
            pl.BlockSpec((_BN, H), lambda i: (i, 0)),
            pl.BlockSpec((_BN, H), lambda i: (i, 0)),
            pl.BlockSpec((_BN, H), lambda i: (i, 0)),
        ],
        out_shape=[
            jax.ShapeDtypeStruct((N, H), f32),
            jax.ShapeDtypeStruct((N, H), f32),
            jax.ShapeDtypeStruct((N, H), f32),
        ],
    )(h, s, d, uw1, uw2, ub, nlw, nlb, mw1, mw2, mb)


def _tc_node_update_last(h, s, d, uw1, uw2, ub, nlw, nlb):
    return pl.pallas_call(
        _node_update_last_body,
        grid=(N // _BN,),
        in_specs=_nu_in_specs(),
        out_specs=pl.BlockSpec((_BN, H), lambda i: (i, 0)),
        out_shape=jax.ShapeDtypeStruct((N, H), jnp.float32),
    )(h, s, d, uw1, uw2, ub, nlw, nlb)


_BP = 400


def _pool_head_body(h_ref, b_ref, hw_ref, hb_ref, out_ref, pooled, cnt):
    i = pl.program_id(0)

    @pl.when(i == 0)
    def _():
        pooled[...] = jnp.zeros_like(pooled)
        cnt[...] = jnp.zeros_like(cnt)

    gids = lax.broadcasted_iota(jnp.int32, (_BP, G), 1)
    onehot = jnp.where(b_ref[...] == gids, 1.0, 0.0).astype(jnp.float32)
    dn = (((0,), (0,)), ((), ()))
    pooled[...] += lax.dot_general(onehot, h_ref[...], dn,
                                   preferred_element_type=jnp.float32)
    cnt[...] += lax.dot_general(onehot, jnp.ones((_BP, H), jnp.float32), dn,
                                preferred_element_type=jnp.float32)

    @pl.when(i == pl.num_programs(0) - 1)
    def _():
        pm = pooled[...] / jnp.maximum(cnt[...], 1.0)
        out_ref[...] = _dot(pm, hw_ref[...]) + hb_ref[...]


def _tc_pool_head(h, batch2d, hw, hb):
    return pl.pallas_call(
        _pool_head_body,
        grid=(N // _BP,),
        in_specs=[
            pl.BlockSpec((_BP, H), lambda i: (i, 0)),
            pl.BlockSpec((_BP, 1), lambda i: (i, 0)),
            pl.BlockSpec((H, 1), lambda i: (0, 0)),
            pl.BlockSpec((1, 1), lambda i: (0, 0)),
        ],
        out_specs=pl.BlockSpec((G, 1), lambda i: (0, 0)),
        out_shape=jax.ShapeDtypeStruct((G, 1), jnp.float32),
        scratch_shapes=[
            pltpu.VMEM((G, H), jnp.float32),
            pltpu.VMEM((G, H), jnp.float32),
        ],
    )(h, batch2d, hw, hb)


def kernel(x, edge_index, edge_attr, batch, node_W, node_b, edge_W, edge_b,
           M_W, M_b, U_W, U_b, nl_W, nl_b, el_W, el_b, head_W, head_b):
    src = edge_index[0]
    dst = edge_index[1]
    mw1 = M_W[:, :H, :]
    mw2 = M_W[:, H:2 * H, :]
    mw3 = M_W[:, 2 * H:, :]
    uw1 = U_W[:, :H, :]
    uw2 = U_W[:, H:, :]
    zrows = jnp.zeros((_ZROWS, H), jnp.float32)

    h, a_tab, b_tab = _tc_node_encode(
        x, node_W, node_b.reshape(1, H), mw1[0], mw2[0], M_b[0].reshape(1, H))
    dpart = _sc_degree(dst, zrows)

    ea, c0 = _tc_edge_encode(edge_attr, edge_W, edge_b.reshape(1, H), mw3[0])
    cs = [c0]
    for i in range(L - 1):
        ea, ci = _tc_edge_update(ea, el_W[i], el_b[i].reshape(1, H),
                                 mw3[i + 1])
        cs.append(ci)

    for i in range(L):
        s = _sc_message(a_tab, b_tab, cs[i].reshape(_NW * _CPT, _K, H),
                        src, dst, zrows)
        if i + 1 < L:
            h, a_tab, b_tab = _tc_node_update(
                h, s, dpart, uw1[i], uw2[i], U_b[i].reshape(1, H),
                nl_W[i], nl_b[i].reshape(1, H),
                mw1[i + 1], mw2[i + 1], M_b[i + 1].reshape(1, H))
        else:
            h = _tc_node_update_last(
                h, s, dpart, uw1[i], uw2[i], U_b[i].reshape(1, H),
                nl_W[i], nl_b[i].reshape(1, H))

    return _tc_pool_head(h, batch.reshape(N, 1), head_W, head_b.reshape(1, 1))
